# Initial kernel scaffold; baseline (speedup 1.0000x reference)
import jax
import jax.numpy as jnp
from jax.experimental import pallas as pl


def kernel(tokens, users, batch_idx, emb_user, emb_item):
    raise NotImplementedError("write your pallas kernel here")


# speedup vs baseline: 1.4571x; 1.4571x over previous
import functools

import jax
import jax.numpy as jnp
from jax import lax
from jax.experimental import pallas as pl
from jax.experimental.pallas import tpu as pltpu
from jax.experimental.pallas import tpu_sc as plsc


def _build(B, S, D, num_users):
    info = plsc.get_sparse_core_info()
    NC, NS = info.num_cores, info.num_subcores
    NW = NC * NS
    assert B % NW == 0
    BPW = B // NW

    mesh = plsc.VectorSubcoreMesh(core_axis_name="c", subcore_axis_name="s")

    @functools.partial(
        pl.kernel,
        mesh=mesh,
        out_type=jax.ShapeDtypeStruct((B, S, 2, D), jnp.float32),
        compiler_params=pltpu.CompilerParams(use_tc_tiling_on_sc=False),
        scratch_types=[
            pltpu.VMEM((BPW, S), jnp.int32),
            pltpu.VMEM((BPW,), jnp.int32),
            pltpu.VMEM((BPW,), jnp.int32),
            pltpu.VMEM((BPW, D), jnp.float32),
            pltpu.VMEM((S, D), jnp.float32),
            pltpu.SemaphoreType.DMA,
        ],
    )
    def k(tokens_h, users_h, emb_user_h, emb_item_h, out_h,
          tidx_v, uraw_v, uidx_v, urows_v, irows_v, gsem):
        wid = lax.axis_index("s") * NC + lax.axis_index("c")
        b0 = wid * BPW

        pltpu.sync_copy(tokens_h.at[pl.ds(b0, BPW)], tidx_v)
        pltpu.sync_copy(users_h.at[pl.ds(b0, BPW)], uraw_v)

        for i in range(BPW // 16):
            u = uraw_v[pl.ds(i * 16, 16)]
            uidx_v[pl.ds(i * 16, 16)] = jnp.where(u == 0, num_users - 1, u - 1)

        pltpu.async_copy(emb_user_h.at[uidx_v], urows_v, gsem).wait()

        def user_body(s, carry):
            pltpu.sync_copy(urows_v, out_h.at[pl.ds(b0, BPW), s, 1])
            return carry
        lax.fori_loop(0, S, user_body, 0)

        def item_body(i, carry):
            pltpu.async_copy(emb_item_h.at[tidx_v.at[i]], irows_v, gsem).wait()
            pltpu.sync_copy(irows_v, out_h.at[b0 + i, :, 0])
            return carry
        lax.fori_loop(0, BPW, item_body, 0)

    return k


def kernel(tokens, users, batch_idx, emb_user, emb_item):
    del batch_idx
    B, S = tokens.shape
    D = emb_item.shape[1]
    tokens = tokens.astype(jnp.int32)
    users = users.astype(jnp.int32)
    out = _build(B, S, D, emb_user.shape[0])(tokens, users, emb_user, emb_item)
    return out.reshape(B, S, 2 * D)


# speedup vs baseline: 1.6064x; 1.1025x over previous
import functools

import jax
import jax.numpy as jnp
from jax import lax
from jax.experimental import pallas as pl
from jax.experimental.pallas import tpu as pltpu
from jax.experimental.pallas import tpu_sc as plsc


def _build(B, S, D, num_users):
    info = plsc.get_sparse_core_info()
    NC, NS, L = info.num_cores, info.num_subcores, info.num_lanes
    NW = NC * NS
    assert B % NW == 0 and D == 64 and (B * S) % 128 == 0
    BPW = B // NW
    RPW = BPW * S
    NCHUNK = RPW // 128

    mesh = plsc.VectorSubcoreMesh(core_axis_name="c", subcore_axis_name="s")

    @functools.partial(
        pl.kernel,
        mesh=mesh,
        out_type=jax.ShapeDtypeStruct((B * S, 2 * D), jnp.float32),
        compiler_params=pltpu.CompilerParams(
            use_tc_tiling_on_sc=False, needs_layout_passes=False),
        scratch_types=[
            pltpu.VMEM((NCHUNK, 128), jnp.int32),
            pltpu.VMEM((BPW,), jnp.int32),
            pltpu.VMEM((BPW,), jnp.int32),
            pltpu.VMEM((RPW,), jnp.int32),
            pltpu.VMEM((128, 2 * D), jnp.float32),
            pltpu.VMEM((128, 2 * D), jnp.float32),
            pltpu.SemaphoreType.DMA,
        ],
    )
    def k(tokens_h, users_h, emb_user_h, emb_item_h, out_h,
          tidx_v, uraw_v, uidx_v, urep_v, ibuf_v, ubuf_v, gsem):
        wid = lax.axis_index("s") * NC + lax.axis_index("c")
        b0 = wid * BPW
        o0 = wid * RPW

        pltpu.sync_copy(tokens_h.at[pl.ds(wid * NCHUNK, NCHUNK)], tidx_v)
        pltpu.sync_copy(users_h.at[pl.ds(b0, BPW)], uraw_v)

        lanes = lax.iota(jnp.int32, L)
        for g in range(BPW // L):
            u = uraw_v[pl.ds(g * L, L)]
            u = jnp.where(u == 0, num_users - 1, u - 1)
            uidx_v[pl.ds(g * L, L)] = u
            pos0 = (g * L + lanes) * S

            def rep_body(s, carry):
                plsc.store_scatter(urep_v, [pos0 + s], u)
                return carry
            lax.fori_loop(0, S, rep_body, 0)

        def chunk_body(c, carry):
            r0 = o0 + c * 128
            pltpu.async_copy(emb_item_h.at[tidx_v.at[c]], ibuf_v, gsem).wait()
            pltpu.sync_copy(ibuf_v.at[:, pl.ds(0, D)],
                            out_h.at[pl.ds(r0, 128), pl.ds(0, D)])
            pltpu.async_copy(emb_user_h.at[urep_v.at[pl.ds(c * 128, 128)]],
                             ubuf_v, gsem).wait()
            pltpu.sync_copy(ubuf_v.at[:, pl.ds(0, D)],
                            out_h.at[pl.ds(r0, 128), pl.ds(D, D)])
            return carry
        lax.fori_loop(0, NCHUNK, chunk_body, 0)

    return k


def kernel(tokens, users, batch_idx, emb_user, emb_item):
    del batch_idx
    B, S = tokens.shape
    D = emb_item.shape[1]
    num_users = emb_user.shape[0]
    tokens2 = tokens.astype(jnp.int32).reshape((B * S) // 128, 128)
    users = users.astype(jnp.int32)
    ni = emb_item.shape[0]
    ni_pad = (-ni) % 8
    emb_item_p = jnp.pad(emb_item, ((0, ni_pad), (0, 128 - D)))
    emb_user_p = jnp.pad(emb_user, ((0, (-num_users) % 8), (0, 128 - D)))
    out = _build(B, S, D, num_users)(tokens2, users, emb_user_p, emb_item_p)
    return out.reshape(B, S, 2 * D)


# speedup vs baseline: 2.9947x; 1.8642x over previous
import functools

import jax
import jax.numpy as jnp
from jax import lax
from jax.experimental import pallas as pl
from jax.experimental.pallas import tpu as pltpu
from jax.experimental.pallas import tpu_sc as plsc


def _build(B, S, D, num_users):
    info = plsc.get_sparse_core_info()
    NC, NS, L = info.num_cores, info.num_subcores, info.num_lanes
    NW = NC * NS
    assert B % NW == 0 and D == 64 and (B * S) % 128 == 0
    BPW = B // NW
    RPW = BPW * S
    NCHUNK = RPW // 128
    assert NCHUNK % 2 == 0

    mesh = plsc.VectorSubcoreMesh(core_axis_name="c", subcore_axis_name="s")

    @functools.partial(
        pl.kernel,
        mesh=mesh,
        out_type=jax.ShapeDtypeStruct((B * S, 2 * D), jnp.float32),
        scratch_types=[
            pltpu.VMEM((RPW,), jnp.int32),
            pltpu.VMEM((BPW,), jnp.int32),
            pltpu.VMEM((BPW,), jnp.int32),
            pltpu.VMEM((BPW, 2 * D), jnp.float32),
            pltpu.VMEM((128, 2 * D), jnp.float32),
            pltpu.VMEM((128, 2 * D), jnp.float32),
            pltpu.SemaphoreType.DMA,
            pltpu.SemaphoreType.DMA,
        ],
    )
    def k(tokens_h, users_h, emb_user_h, emb_item_h, out_h,
          tidx_v, uraw_v, uidx_v, urows_v, buf0_v, buf1_v, sem0, sem1):
        wid = lax.axis_index("s") * NC + lax.axis_index("c")
        b0 = wid * BPW
        o0 = wid * RPW

        pltpu.sync_copy(tokens_h.at[pl.ds(o0, RPW)], tidx_v)
        pltpu.sync_copy(users_h.at[pl.ds(b0, BPW)], uraw_v)

        for g in range(BPW // L):
            u = uraw_v[pl.ds(g * L, L)]
            uidx_v[pl.ds(g * L, L)] = jnp.where(u == 0, num_users - 1, u - 1)
        pltpu.async_copy(emb_user_h.at[uidx_v], urows_v, sem0).wait()

        def start_gather(c, buf, sem):
            pltpu.async_copy(emb_item_h.at[tidx_v.at[pl.ds(c * 128, 128)]],
                             buf, sem)

        def wait_gather(c, buf, sem):
            pltpu.make_async_copy(
                emb_item_h.at[tidx_v.at[pl.ds(c * 128, 128)]], buf, sem).wait()

        def fill_and_write(c, buf):
            def row_body(j, carry):
                i = (c * 128 + j) // S
                for q in range(D // L):
                    buf[j, pl.ds(D + q * L, L)] = urows_v[i, pl.ds(q * L, L)]
                return carry
            lax.fori_loop(0, 128, row_body, 0)
            pltpu.sync_copy(buf, out_h.at[pl.ds(o0 + c * 128, 128)])

        start_gather(0, buf0_v, sem0)

        def pair_body(h, carry):
            c0 = 2 * h
            c1 = 2 * h + 1
            start_gather(c1, buf1_v, sem1)
            wait_gather(c0, buf0_v, sem0)
            fill_and_write(c0, buf0_v)

            @pl.when(h < NCHUNK // 2 - 1)
            def _():
                start_gather(c0 + 2, buf0_v, sem0)
            wait_gather(c1, buf1_v, sem1)
            fill_and_write(c1, buf1_v)
            return carry
        lax.fori_loop(0, NCHUNK // 2, pair_body, 0)

    return k


def kernel(tokens, users, batch_idx, emb_user, emb_item):
    del batch_idx
    B, S = tokens.shape
    D = emb_item.shape[1]
    num_users = emb_user.shape[0]
    tokens1 = tokens.astype(jnp.int32).reshape(B * S)
    users = users.astype(jnp.int32)
    emb_item_p = jnp.pad(emb_item, ((0, (-emb_item.shape[0]) % 8), (0, 128 - D)))
    emb_user_p = jnp.pad(emb_user, ((0, (-num_users) % 8), (0, 128 - D)))
    out = _build(B, S, D, num_users)(tokens1, users, emb_user_p, emb_item_p)
    return out.reshape(B, S, 2 * D)


# speedup vs baseline: 4.2642x; 1.4239x over previous
import functools

import jax
import jax.numpy as jnp
from jax import lax
from jax.experimental import pallas as pl
from jax.experimental.pallas import tpu as pltpu
from jax.experimental.pallas import tpu_sc as plsc


def _pad_tables(emb_user, emb_item, D):
    nu = emb_user.shape[0]
    ni = emb_item.shape[0]
    nu_p = nu + (-nu) % 8
    ni_p = ni + (-ni) % 8
    blk = 4096
    grid = (max(nu_p, ni_p) + blk - 1) // blk

    def body(u_ref, i_ref, up_ref, ip_ref):
        z = jnp.zeros((blk, 128 - D), jnp.float32)
        up_ref[...] = jnp.concatenate([u_ref[...], z], axis=1)
        ip_ref[...] = jnp.concatenate([i_ref[...], z], axis=1)

    return pl.pallas_call(
        body,
        grid=(grid,),
        in_specs=[
            pl.BlockSpec((blk, D), lambda g: (g, 0)),
            pl.BlockSpec((blk, D), lambda g: (g, 0)),
        ],
        out_specs=[
            pl.BlockSpec((blk, 128), lambda g: (g, 0)),
            pl.BlockSpec((blk, 128), lambda g: (g, 0)),
        ],
        out_shape=[
            jax.ShapeDtypeStruct((nu_p, 128), jnp.float32),
            jax.ShapeDtypeStruct((ni_p, 128), jnp.float32),
        ],
    )(emb_user, emb_item)


def _build(B, S, D, num_users):
    info = plsc.get_sparse_core_info()
    NC, NS, L = info.num_cores, info.num_subcores, info.num_lanes
    NW = NC * NS
    assert B % (2 * NW) == 0 and D == 64
    BPW = B // NW
    RPW = BPW * S

    mesh = plsc.VectorSubcoreMesh(core_axis_name="c", subcore_axis_name="s")

    @functools.partial(
        pl.kernel,
        mesh=mesh,
        out_type=jax.ShapeDtypeStruct((B, S, 2 * D), jnp.float32),
        scratch_types=[
            pltpu.VMEM((RPW,), jnp.int32),
            pltpu.VMEM((BPW,), jnp.int32),
            pltpu.VMEM((BPW,), jnp.int32),
            pltpu.VMEM((BPW, 2 * D), jnp.float32),
            pltpu.VMEM((4 * S, 2 * D), jnp.float32),
            pltpu.VMEM((4 * S, 2 * D), jnp.float32),
            pltpu.SemaphoreType.DMA,
            pltpu.SemaphoreType.DMA,
        ],
    )
    def k(tokens_h, users_h, emb_user_h, emb_item_h, out_h,
          tidx_v, uraw_v, uidx_v, urows_v, buf0_v, buf1_v, sem0, sem1):
        wid = lax.axis_index("s") * NC + lax.axis_index("c")
        b0 = wid * BPW
        o0 = wid * RPW
        GS = 4 * S
        NG = BPW // 4

        pltpu.sync_copy(tokens_h.at[pl.ds(o0, RPW)], tidx_v)
        pltpu.sync_copy(users_h.at[pl.ds(b0, BPW)], uraw_v)

        for g in range(BPW // L):
            u = uraw_v[pl.ds(g * L, L)]
            uidx_v[pl.ds(g * L, L)] = jnp.where(u == 0, num_users - 1, u - 1)
        pltpu.async_copy(emb_user_h.at[uidx_v], urows_v, sem0).wait()

        def start_gather(g, buf, sem):
            pltpu.async_copy(emb_item_h.at[tidx_v.at[pl.ds(g * GS, 128)]],
                             buf.at[pl.ds(0, 128)], sem)
            pltpu.async_copy(
                emb_item_h.at[tidx_v.at[pl.ds(g * GS + 128, GS - 128)]],
                buf.at[pl.ds(128, GS - 128)], sem)

        def wait_gather(g, buf, sem):
            pltpu.make_async_copy(
                emb_item_h.at[tidx_v.at[pl.ds(g * GS, 128)]],
                buf.at[pl.ds(0, 128)], sem).wait()
            pltpu.make_async_copy(
                emb_item_h.at[tidx_v.at[pl.ds(g * GS + 128, GS - 128)]],
                buf.at[pl.ds(128, GS - 128)], sem).wait()

        def fill_and_write(g, buf):
            for a in range(4):
                for q in range(D // L):
                    v = urows_v[4 * g + a, pl.ds(q * L, L)]
                    for s in range(S):
                        buf[a * S + s, pl.ds(D + q * L, L)] = v
                pltpu.sync_copy(buf.at[pl.ds(a * S, S)], out_h.at[b0 + 4 * g + a])

        start_gather(0, buf0_v, sem0)

        def pair_body(p, carry):
            g0 = 2 * p
            g1 = 2 * p + 1
            start_gather(g1, buf1_v, sem1)
            wait_gather(g0, buf0_v, sem0)
            fill_and_write(g0, buf0_v)

            @pl.when(p < NG // 2 - 1)
            def _():
                start_gather(g0 + 2, buf0_v, sem0)
            wait_gather(g1, buf1_v, sem1)
            fill_and_write(g1, buf1_v)
            return carry
        lax.fori_loop(0, NG // 2, pair_body, 0)

    return k


def kernel(tokens, users, batch_idx, emb_user, emb_item):
    del batch_idx
    B, S = tokens.shape
    D = emb_item.shape[1]
    num_users = emb_user.shape[0]
    tokens1 = tokens.astype(jnp.int32).reshape(B * S)
    users = users.astype(jnp.int32)
    emb_user_p, emb_item_p = _pad_tables(emb_user, emb_item, D)
    return _build(B, S, D, num_users)(tokens1, users, emb_user_p, emb_item_p)


# speedup vs baseline: 5.6182x; 1.3175x over previous
import functools

import jax
import jax.numpy as jnp
from jax import lax
from jax.experimental import pallas as pl
from jax.experimental.pallas import tpu as pltpu
from jax.experimental.pallas import tpu_sc as plsc


def _pad_tables(emb_user, emb_item, D):
    nu = emb_user.shape[0]
    ni = emb_item.shape[0]
    nu_p = nu + (-nu) % 8
    ni_p = ni + (-ni) % 8
    blk = 4096
    grid = (max(nu_p, ni_p) + blk - 1) // blk

    def body(u_ref, i_ref, up_ref, ip_ref):
        z = jnp.zeros((blk, 128 - D), jnp.float32)
        up_ref[...] = jnp.concatenate([u_ref[...], z], axis=1)
        ip_ref[...] = jnp.concatenate([i_ref[...], z], axis=1)

    return pl.pallas_call(
        body,
        grid=(grid,),
        in_specs=[
            pl.BlockSpec((blk, D), lambda g: (g, 0)),
            pl.BlockSpec((blk, D), lambda g: (g, 0)),
        ],
        out_specs=[
            pl.BlockSpec((blk, 128), lambda g: (g, 0)),
            pl.BlockSpec((blk, 128), lambda g: (g, 0)),
        ],
        out_shape=[
            jax.ShapeDtypeStruct((nu_p, 128), jnp.float32),
            jax.ShapeDtypeStruct((ni_p, 128), jnp.float32),
        ],
    )(emb_user, emb_item)


def _build(B, S, D, num_users):
    info = plsc.get_sparse_core_info()
    NC, NS, L = info.num_cores, info.num_subcores, info.num_lanes
    NW = NC * NS
    assert B % NW == 0 and D == 64 and S % 2 == 0
    BPW = B // NW

    mesh = plsc.VectorSubcoreMesh(core_axis_name="c", subcore_axis_name="s")

    @functools.partial(
        pl.kernel,
        mesh=mesh,
        out_type=jax.ShapeDtypeStruct((S, B, 2 * D), jnp.float32),
        scratch_types=[
            pltpu.VMEM((S, BPW), jnp.int32),
            pltpu.VMEM((BPW,), jnp.int32),
            pltpu.VMEM((BPW,), jnp.int32),
            pltpu.VMEM((BPW, 2 * D), jnp.float32),
            pltpu.VMEM((BPW, 2 * D), jnp.float32),
            pltpu.VMEM((BPW, 2 * D), jnp.float32),
            pltpu.SemaphoreType.DMA,
            pltpu.SemaphoreType.DMA,
        ],
    )
    def k(tokens_h, users_h, emb_user_h, emb_item_h, out_h,
          tidx_v, uraw_v, uidx_v, urows_v, buf0_v, buf1_v, sem0, sem1):
        wid = lax.axis_index("s") * NC + lax.axis_index("c")
        b0 = wid * BPW

        pltpu.sync_copy(tokens_h.at[:, pl.ds(b0, BPW)], tidx_v)
        pltpu.sync_copy(users_h.at[pl.ds(b0, BPW)], uraw_v)

        for g in range(BPW // L):
            u = uraw_v[pl.ds(g * L, L)]
            uidx_v[pl.ds(g * L, L)] = jnp.where(u == 0, num_users - 1, u - 1)
        pltpu.async_copy(emb_user_h.at[uidx_v], urows_v, sem0).wait()

        def start_gather(s, buf, sem):
            pltpu.async_copy(emb_item_h.at[tidx_v.at[s]], buf, sem)

        def wait_gather(s, buf, sem):
            pltpu.make_async_copy(
                emb_item_h.at[tidx_v.at[s]], buf, sem).wait()

        def fill_and_write(s, buf):
            for j in range(BPW):
                for q in range(D // L):
                    buf[j, pl.ds(D + q * L, L)] = urows_v[j, pl.ds(q * L, L)]
            pltpu.sync_copy(buf, out_h.at[s, pl.ds(b0, BPW)])

        start_gather(0, buf0_v, sem0)

        def pair_body(p, carry):
            s0 = 2 * p
            s1 = 2 * p + 1
            start_gather(s1, buf1_v, sem1)
            wait_gather(s0, buf0_v, sem0)
            fill_and_write(s0, buf0_v)

            @pl.when(p < S // 2 - 1)
            def _():
                start_gather(s0 + 2, buf0_v, sem0)
            wait_gather(s1, buf1_v, sem1)
            fill_and_write(s1, buf1_v)
            return carry
        lax.fori_loop(0, S // 2, pair_body, 0)

    return k


def kernel(tokens, users, batch_idx, emb_user, emb_item):
    del batch_idx
    B, S = tokens.shape
    D = emb_item.shape[1]
    num_users = emb_user.shape[0]
    tokens_sm = tokens.astype(jnp.int32).T
    users = users.astype(jnp.int32)
    emb_user_p, emb_item_p = _pad_tables(emb_user, emb_item, D)
    out = _build(B, S, D, num_users)(tokens_sm, users, emb_user_p, emb_item_p)
    return out.transpose(1, 0, 2)


# speedup vs baseline: 7.9731x; 1.4192x over previous
import functools

import jax
import jax.numpy as jnp
from jax import lax
from jax.experimental import pallas as pl
from jax.experimental.pallas import tpu as pltpu
from jax.experimental.pallas import tpu_sc as plsc


def _pad_tables(emb_user_t, emb_item_t, D):
    nu = emb_user_t.shape[1]
    ni = emb_item_t.shape[1]
    nu_p = nu + (-nu) % 8
    ni_p = ni + (-ni) % 8
    blk = 2048
    grid = (max(nu_p, ni_p) + blk - 1) // blk

    def body(u_ref, i_ref, up_ref, ip_ref):
        up_ref[:, pl.ds(0, D)] = u_ref[...].T
        ip_ref[:, pl.ds(0, D)] = i_ref[...].T

    return pl.pallas_call(
        body,
        grid=(grid,),
        in_specs=[
            pl.BlockSpec((D, blk), lambda g: (0, g)),
            pl.BlockSpec((D, blk), lambda g: (0, g)),
        ],
        out_specs=[
            pl.BlockSpec((blk, 128), lambda g: (g, 0)),
            pl.BlockSpec((blk, 128), lambda g: (g, 0)),
        ],
        out_shape=[
            jax.ShapeDtypeStruct((nu_p, 128), jnp.float32),
            jax.ShapeDtypeStruct((ni_p, 128), jnp.float32),
        ],
    )(emb_user_t, emb_item_t)


def _build(B, S, D, num_users):
    info = plsc.get_sparse_core_info()
    NC, NS, L = info.num_cores, info.num_subcores, info.num_lanes
    NW = NC * NS
    assert B % NW == 0 and D == 64 and S % 2 == 0
    BPW = B // NW

    mesh = plsc.VectorSubcoreMesh(core_axis_name="c", subcore_axis_name="s")

    @functools.partial(
        pl.kernel,
        mesh=mesh,
        out_type=jax.ShapeDtypeStruct((S, B, 2 * D), jnp.float32),
        scratch_types=[
            pltpu.VMEM((S, BPW), jnp.int32),
            pltpu.VMEM((BPW,), jnp.int32),
            pltpu.VMEM((BPW,), jnp.int32),
            pltpu.VMEM((BPW, 2 * D), jnp.float32),
            pltpu.VMEM((BPW, 2 * D), jnp.float32),
            pltpu.VMEM((BPW, 2 * D), jnp.float32),
            pltpu.SemaphoreType.DMA,
            pltpu.SemaphoreType.DMA,
        ],
    )
    def k(tokens_h, users_h, emb_user_h, emb_item_h, out_h,
          tidx_v, uraw_v, uidx_v, urows_v, buf0_v, buf1_v, sem0, sem1):
        wid = lax.axis_index("s") * NC + lax.axis_index("c")
        b0 = wid * BPW

        pltpu.sync_copy(tokens_h.at[:, pl.ds(b0, BPW)], tidx_v)
        pltpu.sync_copy(users_h.at[pl.ds(b0, BPW)], uraw_v)

        for g in range(BPW // L):
            u = uraw_v[pl.ds(g * L, L)]
            uidx_v[pl.ds(g * L, L)] = jnp.where(u == 0, num_users - 1, u - 1)
        pltpu.async_copy(emb_user_h.at[uidx_v], urows_v, sem0).wait()

        def start_gather(s, buf, sem):
            pltpu.async_copy(emb_item_h.at[tidx_v.at[s]], buf, sem)

        def wait_gather(s, buf, sem):
            pltpu.make_async_copy(
                emb_item_h.at[tidx_v.at[s]], buf, sem).wait()

        def fill_and_write(s, buf):
            for j in range(BPW):
                for q in range(D // L):
                    buf[j, pl.ds(D + q * L, L)] = urows_v[j, pl.ds(q * L, L)]
            pltpu.sync_copy(buf, out_h.at[s, pl.ds(b0, BPW)])

        start_gather(0, buf0_v, sem0)

        def pair_body(p, carry):
            s0 = 2 * p
            s1 = 2 * p + 1
            start_gather(s1, buf1_v, sem1)
            wait_gather(s0, buf0_v, sem0)
            fill_and_write(s0, buf0_v)

            @pl.when(p < S // 2 - 1)
            def _():
                start_gather(s0 + 2, buf0_v, sem0)
            wait_gather(s1, buf1_v, sem1)
            fill_and_write(s1, buf1_v)
            return carry
        lax.fori_loop(0, S // 2, pair_body, 0)

    return k


def kernel(tokens, users, batch_idx, emb_user, emb_item):
    del batch_idx
    B, S = tokens.shape
    D = emb_item.shape[1]
    num_users = emb_user.shape[0]
    tokens_sm = tokens.astype(jnp.int32).T
    users = users.astype(jnp.int32)
    emb_user_p, emb_item_p = _pad_tables(emb_user.T, emb_item.T, D)
    out = _build(B, S, D, num_users)(tokens_sm, users, emb_user_p, emb_item_p)
    return out.transpose(1, 0, 2)
